# Initial kernel scaffold; baseline (speedup 1.0000x reference)
import jax
import jax.numpy as jnp
from jax.experimental import pallas as pl


def kernel(inputs, W, b):
    raise NotImplementedError("write your pallas kernel here")


# speedup vs baseline: 4.9939x; 4.9939x over previous
import jax
import jax.numpy as jnp
from jax.experimental import pallas as pl
from jax.experimental.pallas import tpu as pltpu

_IN = 1024
_OUT = 32768
_K = 256
_B = 8
_BLOCK_N = 2048
_NBLK = _OUT // _BLOCK_N


def _float_key(t):
    i = jax.lax.bitcast_convert_type(t, jnp.int32)
    return jnp.where(i >= 0, i, i ^ jnp.int32(0x7FFFFFFF))


def _wta_kernel(x_ref, w_ref, b_ref, out_ref, t_ref):
    step = pl.program_id(0)
    t_blk = jax.lax.dot_general(
        x_ref[...], w_ref[...],
        (((1,), (1,)), ((), ())),
        preferred_element_type=jnp.float32,
    ) + b_ref[...]
    t_ref[:, pl.ds(step * _BLOCK_N, _BLOCK_N)] = t_blk

    @pl.when(step == _NBLK - 1)
    def _():
        t = t_ref[...]
        key = _float_key(t)
        msb = jnp.int32(-2147483648)

        def kth_body(j, prefix_u):
            bit = 31 - j
            cand_u = prefix_u | (jnp.int32(1) << bit)
            cand_s = cand_u ^ msb
            cnt = jnp.sum((key >= cand_s).astype(jnp.int32), axis=1,
                          keepdims=True)
            return jnp.where(cnt >= _K, cand_u, prefix_u)

        prefix_u = jax.lax.fori_loop(
            0, 32, kth_body, jnp.zeros((_B, 1), jnp.int32))
        thresh = prefix_u ^ msb

        gt = key > thresh
        eq = key == thresh
        r = _K - jnp.sum(gt.astype(jnp.int32), axis=1, keepdims=True)
        col = jax.lax.broadcasted_iota(jnp.int32, (_B, _OUT), 1)

        def idx_body(j, mpref):
            bit = 15 - j
            cand = mpref | (jnp.int32(1) << bit)
            cnt = jnp.sum((eq & (col < cand)).astype(jnp.int32), axis=1,
                          keepdims=True)
            return jnp.where(cnt <= r, cand, mpref)

        mbound = jax.lax.fori_loop(
            0, 16, idx_body, jnp.zeros((_B, 1), jnp.int32))

        accept = gt | (eq & (col < mbound))
        neg = jnp.float32(-jnp.inf)
        pooled = jnp.max(jnp.where(accept, t, neg), axis=0, keepdims=True)
        out_ref[...] = jnp.where(pooled == neg, jnp.float32(0.0), pooled)


def kernel(inputs, W, b):
    out = pl.pallas_call(
        _wta_kernel,
        grid=(_NBLK,),
        in_specs=[
            pl.BlockSpec((_B, _IN), lambda i: (0, 0)),
            pl.BlockSpec((_BLOCK_N, _IN), lambda i: (i, 0)),
            pl.BlockSpec((1, _BLOCK_N), lambda i: (0, i)),
        ],
        out_specs=pl.BlockSpec((1, _OUT), lambda i: (0, 0)),
        out_shape=jax.ShapeDtypeStruct((1, _OUT), jnp.float32),
        scratch_shapes=[pltpu.VMEM((_B, _OUT), jnp.float32)],
    )(inputs, W, b.reshape(1, _OUT))
    return out.reshape(_OUT)


# speedup vs baseline: 5.0229x; 1.0058x over previous
import jax
import jax.numpy as jnp
from jax.experimental import pallas as pl
from jax.experimental.pallas import tpu as pltpu

_IN = 1024
_OUT = 32768
_K = 256
_B = 8
_BLOCK_N = 2048
_NBLK = _OUT // _BLOCK_N


def _float_key(t):
    i = jax.lax.bitcast_convert_type(t, jnp.int32)
    return jnp.where(i >= 0, i, i ^ jnp.int32(0x7FFFFFFF))


def _wta_kernel(x_ref, w_ref, b_ref, out_ref, t_ref):
    step = pl.program_id(0)
    t_blk = jax.lax.dot_general(
        x_ref[...], w_ref[...],
        (((1,), (1,)), ((), ())),
        preferred_element_type=jnp.float32,
    ) + b_ref[...]
    t_ref[:, pl.ds(step * _BLOCK_N, _BLOCK_N)] = t_blk

    @pl.when(step == _NBLK - 1)
    def _():
        t = t_ref[...]
        key = _float_key(t)

        prefix_u = jnp.zeros((_B, 1), jnp.int32)
        for bit in range(31, -1, -1):
            bitval = (1 << bit) if bit < 31 else -(1 << 31)
            cand_u = prefix_u | jnp.int32(bitval)
            cand_s = cand_u ^ jnp.int32(-2147483648)
            cnt = jnp.sum((key >= cand_s).astype(jnp.int32), axis=1,
                          keepdims=True)
            prefix_u = jnp.where(cnt >= _K, cand_u, prefix_u)
        thresh = prefix_u ^ jnp.int32(-2147483648)

        gt = key > thresh
        eq = key == thresh
        n_ge = jnp.sum((gt | eq).astype(jnp.int32), axis=1, keepdims=True)

        r = _K - (n_ge - jnp.sum(eq.astype(jnp.int32), axis=1,
                                 keepdims=True))
        col = jax.lax.broadcasted_iota(jnp.int32, (_B, _OUT), 1)
        mpref = jnp.zeros((_B, 1), jnp.int32)
        for bit in range(15, -1, -1):
            cand = mpref | jnp.int32(1 << bit)
            cntc = jnp.sum((eq & (col < cand)).astype(jnp.int32), axis=1,
                           keepdims=True)
            mpref = jnp.where(cntc <= r, cand, mpref)

        accept = gt | (eq & (col < mpref))
        neg = jnp.float32(-jnp.inf)
        pooled = jnp.max(jnp.where(accept, t, neg), axis=0, keepdims=True)
        out_ref[...] = jnp.where(pooled == neg, jnp.float32(0.0), pooled)


def kernel(inputs, W, b):
    out = pl.pallas_call(
        _wta_kernel,
        grid=(_NBLK,),
        in_specs=[
            pl.BlockSpec((_B, _IN), lambda i: (0, 0)),
            pl.BlockSpec((_BLOCK_N, _IN), lambda i: (i, 0)),
            pl.BlockSpec((1, _BLOCK_N), lambda i: (0, i)),
        ],
        out_specs=pl.BlockSpec((1, _OUT), lambda i: (0, 0)),
        out_shape=jax.ShapeDtypeStruct((1, _OUT), jnp.float32),
        scratch_shapes=[pltpu.VMEM((_B, _OUT), jnp.float32)],
    )(inputs, W, b.reshape(1, _OUT))
    return out.reshape(_OUT)


# speedup vs baseline: 5.4923x; 1.0934x over previous
import jax
import jax.numpy as jnp
from jax.experimental import pallas as pl
from jax.experimental.pallas import tpu as pltpu

_IN = 1024
_OUT = 32768
_K = 256
_B = 8
_BLOCK_N = 2048
_NBLK = _OUT // _BLOCK_N


def _float_key(t):
    i = jax.lax.bitcast_convert_type(t, jnp.int32)
    return jnp.where(i >= 0, i, i ^ jnp.int32(0x7FFFFFFF))


def _wta_kernel(x_ref, w_ref, b_ref, out_ref, t_ref, mb_ref):
    step = pl.program_id(0)
    t_blk = jax.lax.dot_general(
        x_ref[...], w_ref[...],
        (((1,), (1,)), ((), ())),
        preferred_element_type=jnp.float32,
    ) + b_ref[...]
    t_ref[:, pl.ds(step * _BLOCK_N, _BLOCK_N)] = t_blk

    @pl.when(step == _NBLK - 1)
    def _():
        t = t_ref[...]
        key = _float_key(t)

        prefix_u = jnp.zeros((_B, 1), jnp.int32)
        for bit in range(31, -1, -1):
            bitval = (1 << bit) if bit < 31 else -(1 << 31)
            cand_u = prefix_u | jnp.int32(bitval)
            cand_s = cand_u ^ jnp.int32(-2147483648)
            cnt = jnp.sum((key >= cand_s).astype(jnp.int32), axis=1,
                          keepdims=True)
            prefix_u = jnp.where(cnt >= _K, cand_u, prefix_u)
        thresh = prefix_u ^ jnp.int32(-2147483648)

        gt = key > thresh
        eq = key == thresh
        n_ge = jnp.sum((gt | eq).astype(jnp.int32), axis=1, keepdims=True)

        col = jax.lax.broadcasted_iota(jnp.int32, (_B, _OUT), 1)
        mb_ref[...] = jnp.full((_B, 128), _OUT, jnp.int32)

        @pl.when(jnp.any(n_ge > _K))
        def _tie():
            r = _K - (n_ge - jnp.sum(eq.astype(jnp.int32), axis=1,
                                     keepdims=True))
            mpref = jnp.zeros((_B, 1), jnp.int32)
            for bit in range(15, -1, -1):
                cand = mpref | jnp.int32(1 << bit)
                cntc = jnp.sum((eq & (col < cand)).astype(jnp.int32),
                               axis=1, keepdims=True)
                mpref = jnp.where(cntc <= r, cand, mpref)
            mb_ref[...] = jnp.broadcast_to(mpref, (_B, 128))

        accept = gt | (eq & (col < mb_ref[:, :1]))
        neg = jnp.float32(-jnp.inf)
        pooled = jnp.max(jnp.where(accept, t, neg), axis=0, keepdims=True)
        out_ref[...] = jnp.where(pooled == neg, jnp.float32(0.0), pooled)


def kernel(inputs, W, b):
    out = pl.pallas_call(
        _wta_kernel,
        grid=(_NBLK,),
        in_specs=[
            pl.BlockSpec((_B, _IN), lambda i: (0, 0)),
            pl.BlockSpec((_BLOCK_N, _IN), lambda i: (i, 0)),
            pl.BlockSpec((1, _BLOCK_N), lambda i: (0, i)),
        ],
        out_specs=pl.BlockSpec((1, _OUT), lambda i: (0, 0)),
        out_shape=jax.ShapeDtypeStruct((1, _OUT), jnp.float32),
        scratch_shapes=[pltpu.VMEM((_B, _OUT), jnp.float32),
                        pltpu.VMEM((_B, 128), jnp.int32)],
    )(inputs, W, b.reshape(1, _OUT))
    return out.reshape(_OUT)
